# Initial kernel scaffold; baseline (speedup 1.0000x reference)
import functools

import jax
import jax.numpy as jnp
from jax import lax
from jax.experimental import pallas as pl
from jax.experimental.pallas import tpu as pltpu
from jax.experimental.pallas import tpu_sc as plsc

N = 16384
H = 100
D = 64
L = 16
NLANES = D // L

NC, NS = 2, 16
NW = NC * NS
S_PER_W = N // NW
PAIRS_PER_W = S_PER_W // 2

PB = 2
NBATCH = PAIRS_PER_W // PB

_mesh = plsc.VectorSubcoreMesh(core_axis_name="c", subcore_axis_name="s")


@functools.partial(
    pl.kernel,
    out_type=jax.ShapeDtypeStruct((N, D), jnp.float32),
    mesh=_mesh,
    scratch_types=[
        pltpu.VMEM((PB, 2, H), jnp.int32),
        pltpu.VMEM((PB, 2, H, D), jnp.float32),
        pltpu.VMEM((PB * 2, D), jnp.float32),
        pltpu.VMEM((D,), jnp.float32),
        pltpu.SemaphoreType.DMA,
    ],
)
def _sc_embed_sum(x_hbm, w_hbm, b_hbm, out_hbm, idx_v, rows_v, out_v, bias_v, sem):
    cid = lax.axis_index("c")
    sid = lax.axis_index("s")
    wid = sid * NC + cid

    pltpu.sync_copy(b_hbm, bias_v)
    bias_regs = tuple(bias_v[pl.ds(L * k, L)] for k in range(NLANES))

    pair_base = wid * PAIRS_PER_W

    def batch_body(g, carry):
        p0 = pair_base + g * PB
        pltpu.sync_copy(x_hbm.at[pl.ds(p0, PB)], idx_v)

        copies = [
            pltpu.async_copy(w_hbm.at[idx_v.at[p]], rows_v.at[p], sem)
            for p in range(PB)
        ]
        for cp in copies:
            cp.wait()

        for p in range(PB):
            for h in range(2):
                def red_body(r, accs, _p=p, _h=h):
                    return tuple(
                        accs[k] + rows_v[_p, _h, r, pl.ds(L * k, L)]
                        for k in range(NLANES)
                    )
                accs = lax.fori_loop(0, H, red_body, bias_regs)
                for k in range(NLANES):
                    out_v[2 * p + h, pl.ds(L * k, L)] = accs[k]

        pltpu.sync_copy(out_v, out_hbm.at[pl.ds(2 * p0, 2 * PB)])
        return carry

    lax.fori_loop(0, NBATCH, batch_body, 0)


def kernel(X, W, b):
    X_pairs = X.reshape(N // 2, 2, H)
    return _sc_embed_sum(X_pairs, W, b)


# speedup vs baseline: 4.8842x; 4.8842x over previous
import functools

import jax
import jax.numpy as jnp
from jax import lax
from jax.experimental import pallas as pl
from jax.experimental.pallas import tpu as pltpu
from jax.experimental.pallas import tpu_sc as plsc

N = 16384
H = 100
D = 64
L = 16
NLANES = D // L

NC, NS = 2, 16
NW = NC * NS
S_PER_W = N // NW
PAIRS_PER_W = S_PER_W // 2

HP = 104
SB = 4
NBATCH = S_PER_W // SB

_mesh = plsc.VectorSubcoreMesh(core_axis_name="c", subcore_axis_name="s")


@functools.partial(
    pl.kernel,
    out_type=jax.ShapeDtypeStruct((N, D), jnp.float32),
    mesh=_mesh,
    compiler_params=pltpu.CompilerParams(use_tc_tiling_on_sc=False),
    scratch_types=[
        pltpu.VMEM((SB, HP), jnp.int32),
        pltpu.VMEM((SB, HP, D), jnp.float32),
        pltpu.VMEM((SB, D), jnp.float32),
        pltpu.VMEM((D,), jnp.float32),
        pltpu.SemaphoreType.DMA,
    ],
)
def _sc_embed_sum(x_hbm, w_hbm, b_hbm, out_hbm, idx_v, rows_v, out_v, bias_v, sem):
    cid = lax.axis_index("c")
    sid = lax.axis_index("s")
    wid = sid * NC + cid

    pltpu.sync_copy(b_hbm, bias_v)
    bias_regs = tuple(bias_v[pl.ds(L * k, L)] for k in range(NLANES))

    sample_base = wid * S_PER_W

    def batch_body(g, carry):
        s0 = sample_base + g * SB
        pltpu.sync_copy(x_hbm.at[pl.ds(s0, SB)], idx_v)

        copies = [
            pltpu.async_copy(w_hbm.at[idx_v.at[j]], rows_v.at[j], sem)
            for j in range(SB)
        ]
        for cp in copies:
            cp.wait()

        for j in range(SB):
            def red_body(r, accs, _j=j):
                return tuple(
                    accs[k] + rows_v[_j, r, pl.ds(L * k, L)]
                    for k in range(NLANES)
                )
            accs = lax.fori_loop(0, H, red_body, bias_regs)
            for k in range(NLANES):
                out_v[j, pl.ds(L * k, L)] = accs[k]

        pltpu.sync_copy(out_v, out_hbm.at[pl.ds(s0, SB)])
        return carry

    lax.fori_loop(0, NBATCH, batch_body, 0)


def kernel(X, W, b):
    X_pad = jnp.pad(X, ((0, 0), (0, HP - H)))
    return _sc_embed_sum(X_pad, W, b)
